# Initial kernel scaffold; baseline (speedup 1.0000x reference)
import jax
import jax.numpy as jnp
from jax.experimental import pallas as pl


def kernel(lprobs, scores, finished):
    raise NotImplementedError("write your pallas kernel here")


# speedup vs baseline: 50.3466x; 50.3466x over previous
import functools

import jax
import jax.numpy as jnp
from jax import lax
from jax.experimental import pallas as pl
from jax.experimental.pallas import tpu as pltpu
from jax.experimental.pallas import tpu_sc as plsc

_BEAM = 64
_VOCAB = 100000
_PAD = 0
_EOS = 1
_NINF = float("-inf")
_IMAX = 2**31 - 1

_NC = 2
_NS = 16
_NW = _NC * _NS
_RPW = _BEAM // _NW
_W = 20000
_NWIN = _VOCAB // _W
_GROUP = 10
_NGRP = _W // (16 * _GROUP)
_CAP = 256
_BUF = 432
_OUTC = 256


def _splat_f32(x):
  return jnp.zeros((16,), jnp.float32) + x


def _splat_i32(x):
  return jnp.zeros((16,), jnp.int32) + x


def _iota16():
  return lax.broadcasted_iota(jnp.int32, (16,), 0)


def _sc_scan_body(lp, sp, fp, outv, outi,
                  buf0, buf1, candv, candi, svec, fvec, sem0, sem1):
  wid = lax.axis_index("s") * _NC + lax.axis_index("c")
  iota = _iota16()
  bufs = (buf0, buf1)
  sems = (sem0, sem1)

  def compact(cnt, g):
    candv[pl.ds(cnt, 16)] = _splat_f32(_NINF)
    nv = cnt // 16
    def lb_body(j, acc):
      v = candv[pl.ds(j * 16, 16)]
      sk, _ = plsc.sort_key_val(v, v)
      return jnp.minimum(acc, sk)
    acc = lax.fori_loop(0, nv, lb_body, _splat_f32(jnp.inf))
    lb_s = jnp.sum(acc * (iota == 12).astype(jnp.float32))
    lb = _splat_f32(lb_s)
    nv2 = (cnt + 15) // 16
    def f_body(j, nc):
      v = candv[pl.ds(j * 16, 16)]
      ii = candi[pl.ds(j * 16, 16)]
      m = v >= lb
      plsc.store_compressed(candv.at[pl.ds(nc, 16)], v, mask=m)
      plsc.store_compressed(candi.at[pl.ds(nc, 16)], ii, mask=m)
      return nc + jnp.max(plsc.all_reduce_population_count(m))
    nc = lax.fori_loop(0, nv2, f_body, jnp.int32(0))
    return nc, jnp.maximum(g, lb)

  def keep(cnt, g):
    return cnt, g

  def make_group(cur, idx0, s_vec):
    def group(i, carry):
      cnt, g = carry
      base = i * (16 * _GROUP)
      lt = g - s_vec
      xs = [cur[pl.ds(base + 16 * k, 16)] for k in range(_GROUP)]
      ms = [x > lt for x in xs]
      anyv = functools.reduce(jnp.logical_or, ms)
      def slow(cnt, g):
        for k in range(_GROUP):
          plsc.store_compressed(candv.at[pl.ds(cnt, 16)], xs[k] + s_vec,
                                mask=ms[k])
          idxv = _splat_i32(idx0 + base + 16 * k) + iota
          plsc.store_compressed(candi.at[pl.ds(cnt, 16)], idxv, mask=ms[k])
          cnt = cnt + jnp.max(plsc.all_reduce_population_count(ms[k]))
        return lax.cond(cnt >= _CAP, compact, keep, cnt, g)
      return lax.cond(jnp.any(anyv), slow, keep, cnt, g)
    return group

  def scan_row(row, s_vec, cnt, g):
    row_base = pl.multiple_of(row * _VOCAB, 8)
    descs = [pltpu.async_copy(lp.at[pl.ds(row_base, _W)], bufs[0], sems[0])]
    for j in range(_NWIN):
      if j + 1 < _NWIN:
        descs.append(pltpu.async_copy(
            lp.at[pl.ds(row_base + (j + 1) * _W, _W)],
            bufs[(j + 1) % 2], sems[(j + 1) % 2]))
      descs[j].wait()
      cur = bufs[j % 2]
      if j == 0:
        first = cur[pl.ds(0, 16)]
        cur[pl.ds(0, 16)] = jnp.where(iota == _PAD, _splat_f32(_NINF), first)
      cnt, g = lax.fori_loop(0, _NGRP,
                             make_group(cur, row_base + j * _W, s_vec),
                             (cnt, g))
    return cnt, g

  cnt = jnp.int32(0)
  g = _splat_f32(_NINF)
  for r in range(_RPW):
    row = wid * _RPW + r
    pltpu.sync_copy(sp.at[row], svec)
    pltpu.sync_copy(fp.at[row], fvec)
    s_vec = svec[...]
    finp = jnp.max(plsc.all_reduce_population_count(fvec[...] > 0)) > 0

    def eos_fn(cnt, g, s_vec=s_vec, row=row):
      m0 = iota == 0
      plsc.store_compressed(candv.at[pl.ds(cnt, 16)], s_vec, mask=m0)
      plsc.store_compressed(candi.at[pl.ds(cnt, 16)],
                            _splat_i32(row * _VOCAB + _EOS), mask=m0)
      return cnt + 1, g

    def scan_fn(cnt, g, s_vec=s_vec, row=row):
      return scan_row(row, s_vec, cnt, g)

    cnt, g = lax.cond(finp, eos_fn, scan_fn, cnt, g)

  for _ in range(2):
    cnt, g = lax.cond(cnt > _OUTC, compact, keep, cnt, g)
  for j in range(_OUTC // 16):
    lanes = _splat_i32(16 * j) + iota
    kp = lanes < cnt
    v = candv[pl.ds(16 * j, 16)]
    ii = candi[pl.ds(16 * j, 16)]
    candv[pl.ds(16 * j, 16)] = jnp.where(kp, v, _splat_f32(_NINF))
    candi[pl.ds(16 * j, 16)] = jnp.where(kp, ii, _splat_i32(_IMAX))
  pltpu.sync_copy(candv.at[pl.ds(0, _OUTC)], outv.at[wid])
  pltpu.sync_copy(candi.at[pl.ds(0, _OUTC)], outi.at[wid])


def _merge_body(v_ref, i_ref, os_ref, ot_ref, oo_ref):
  vals0 = v_ref[...]
  idxs = i_ref[...]
  col = lax.broadcasted_iota(jnp.int32, (1, 128), 1)

  def body(i, carry):
    vals, sa, ta, oa = carry
    m = jnp.max(vals)
    sel = vals == m
    ci = jnp.min(jnp.where(sel, idxs, _IMAX))
    vals = jnp.where(sel & (idxs == ci), _NINF, vals)
    sa = jnp.where(col == i, m, sa)
    ta = jnp.where(col == i, ci % _VOCAB, ta)
    oa = jnp.where(col == i, ci // _VOCAB, oa)
    return vals, sa, ta, oa

  init = (vals0,
          jnp.full((1, 128), _NINF, jnp.float32),
          jnp.zeros((1, 128), jnp.int32),
          jnp.zeros((1, 128), jnp.int32))
  _, sa, ta, oa = lax.fori_loop(0, _BEAM, body, init)
  os_ref[...] = sa
  ot_ref[...] = ta
  oo_ref[...] = oa


def _sc_scan(lp_flat, spad, fpad):
  mesh = plsc.VectorSubcoreMesh(core_axis_name="c", subcore_axis_name="s",
                                num_cores=_NC, num_subcores=_NS)
  f = pl.kernel(
      _sc_scan_body,
      out_type=(jax.ShapeDtypeStruct((_NW, _OUTC), jnp.float32),
                jax.ShapeDtypeStruct((_NW, _OUTC), jnp.int32)),
      mesh=mesh,
      compiler_params=pltpu.CompilerParams(needs_layout_passes=False),
      scratch_types=[
          pltpu.VMEM((_W,), jnp.float32),
          pltpu.VMEM((_W,), jnp.float32),
          pltpu.VMEM((_BUF,), jnp.float32),
          pltpu.VMEM((_BUF,), jnp.int32),
          pltpu.VMEM((16,), jnp.float32),
          pltpu.VMEM((16,), jnp.int32),
          pltpu.SemaphoreType.DMA,
          pltpu.SemaphoreType.DMA,
      ],
  )
  return f(lp_flat, spad, fpad)


def _merge(cand_v, cand_i):
  return pl.pallas_call(
      _merge_body,
      out_shape=(jax.ShapeDtypeStruct((1, 128), jnp.float32),
                 jax.ShapeDtypeStruct((1, 128), jnp.int32),
                 jax.ShapeDtypeStruct((1, 128), jnp.int32)),
  )(cand_v.reshape(_BEAM, 128), cand_i.reshape(_BEAM, 128))


def kernel(lprobs, scores, finished):
  lp_flat = lprobs.reshape(-1)
  spad = jnp.broadcast_to(scores.reshape(_BEAM, 1).astype(jnp.float32),
                          (_BEAM, 16))
  fpad = jnp.broadcast_to(finished.astype(jnp.int32).reshape(_BEAM, 1),
                          (_BEAM, 16))
  cand_v, cand_i = _sc_scan(lp_flat, spad, fpad)
  ts, tok, order = _merge(cand_v, cand_i)
  return ts[0, :_BEAM], tok[0, :_BEAM], order[0, :_BEAM]


# speedup vs baseline: 69.1532x; 1.3735x over previous
import jax
import jax.numpy as jnp
from jax import lax
from jax.experimental import pallas as pl
from jax.experimental.pallas import tpu as pltpu
from jax.experimental.pallas import tpu_sc as plsc

_BEAM = 64
_VOCAB = 100000
_PAD = 0
_EOS = 1
_NINF = float("-inf")
_IMAX = 2**31 - 1

_NC = 2
_NS = 16
_NW = _NC * _NS
_NRB = 8
_QPB = 4
_TPW = 196
_TWIN = 28
_NWIN = _TPW // _TWIN
_WCOLS = _TWIN * 128
_SPAN = 4
_NSP = _TWIN // _SPAN
_NGRP = _NSP * 8
_GCOLS = _SPAN * 128
_QROWS = 128
_QTRIG = 24
_FLAT = 2080
_CAP = 256
_OUTC = 256


def _splat_f32(x):
  return jnp.zeros((16,), jnp.float32) + x


def _splat_i32(x):
  return jnp.zeros((16,), jnp.int32) + x


def _iota16():
  return lax.broadcasted_iota(jnp.int32, (16,), 0)


def _sc_scan_body(lp, sp, fp, outv, outi,
                  buf0, buf1, tailb, gm, uref, st8, fv8, qv, qi, candv, candi,
                  sortedv, sortedi, sem0, sem1):
  wid = lax.axis_index("s") * _NC + lax.axis_index("c")
  rb = wid // _QPB
  q = wid % _QPB
  iota = _iota16()
  bufs = (buf0, buf1)
  sems = (sem0, sem1)
  one = _splat_i32(1)
  zero = _splat_i32(0)
  qvec = _splat_i32(0) + q
  is_q0 = qvec == 0
  is_q3 = qvec == 3
  toff = jnp.where(q == _QPB - 1, 585, q * _TPW)
  col0 = pl.multiple_of(toff * 128, 128)

  def lane12(acc):
    acc = jnp.maximum(acc, _splat_f32(-3e38))
    return jnp.sum(acc * (iota == 12).astype(jnp.float32))

  def compact_q(qcnt, g):
    maxq = jnp.max(qcnt)
    ninf = _splat_f32(_NINF)

    def top4_body(j, carry):
      m1, m2, m3, m4 = carry
      v = jnp.where(j < qcnt, qv[j], ninf)
      t2 = jnp.minimum(m1, v)
      m1 = jnp.maximum(m1, v)
      t3 = jnp.minimum(m2, t2)
      m2 = jnp.maximum(m2, t2)
      t4 = jnp.minimum(m3, t3)
      m3 = jnp.maximum(m3, t3)
      m4 = jnp.maximum(m4, t4)
      return m1, m2, m3, m4

    _, _, _, m4 = lax.fori_loop(0, maxq, top4_body, (ninf, ninf, ninf, ninf))
    lb = _splat_f32(jnp.min(m4))

    def filt_body(j, nq):
      v = qv[j]
      ii = qi[j]
      keep = (v >= lb) & (j < qcnt)
      rows = jnp.minimum(nq, _QROWS - 1)
      plsc.store_scatter(qv, [rows, iota], v, mask=keep)
      plsc.store_scatter(qi, [rows, iota], ii, mask=keep)
      return nq + jnp.where(keep, one, zero)

    nq = lax.fori_loop(0, maxq, filt_body, zero)
    return nq, jnp.maximum(g, lb)

  def keep2(qcnt, g):
    return qcnt, g

  rbase = pl.multiple_of(rb * 128, 8)
  pltpu.sync_copy(sp.at[pl.ds(rbase, 128)], st8)
  pltpu.sync_copy(fp.at[pl.ds(rbase, 128)], fv8)
  for ri in range(8):
    s_ = st8[pl.ds(16 * ri, 16)]
    f_ = fv8[pl.ds(16 * ri, 16)]
    uref[pl.ds(16 * ri, 16)] = jnp.where(f_ > 0, _splat_f32(_NINF), s_)

  qcnt = zero
  g = _splat_f32(_NINF)

  for ri in range(8):
    s_ = st8[pl.ds(16 * ri, 16)]
    f_ = fv8[pl.ds(16 * ri, 16)]
    m = (iota == 0) & (f_ > 0) & is_q0
    rows = jnp.minimum(qcnt, _QROWS - 1)
    plsc.store_scatter(qv, [rows, iota], s_, mask=m)
    plsc.store_scatter(qi, [rows, iota],
                       _splat_i32((rb * 8 + ri) * _VOCAB + _EOS), mask=m)
    qcnt = qcnt + jnp.where(m, one, zero)

  rowbase8 = pl.multiple_of(rb * 8, 8)
  descs = [pltpu.async_copy(
      lp.at[pl.ds(rowbase8, 8), pl.ds(col0, _WCOLS)], bufs[0], sems[0])]
  for w in range(_NWIN):
    if w + 1 < _NWIN:
      descs.append(pltpu.async_copy(
          lp.at[pl.ds(rowbase8, 8), pl.ds(col0 + (w + 1) * _WCOLS, _WCOLS)],
          bufs[(w + 1) % 2], sems[(w + 1) % 2]))
    descs[w].wait()
    cur = bufs[w % 2]

    if w == 0:
      ninf0 = _splat_f32(_NINF)

      @pl.when(q == 0)
      def _():
        for ri in range(8):
          plsc.store_scatter(cur, [_splat_i32(ri), iota], ninf0,
                             mask=(iota == _PAD))

      @pl.when(q == _QPB - 1)
      def _():
        def q3m(ri, _c):
          for kk in range(24):
            cur[ri, pl.ds(16 * kk, 16)] = ninf0
          return 0
        lax.fori_loop(0, 8, q3m, 0)
      ninf = _splat_f32(_NINF)
      m1 = ninf
      m2 = ninf
      m3 = ninf
      m4 = ninf
      for ri in range(8):
        u_ = uref[pl.ds(16 * ri, 16)]
        for kk in range(2):
          x = cur[ri, pl.ds(3 * 128 + 16 * kk, 16)] + u_
          t2 = jnp.minimum(m1, x)
          m1 = jnp.maximum(m1, x)
          t3 = jnp.minimum(m2, t2)
          m2 = jnp.maximum(m2, t2)
          t4 = jnp.minimum(m3, t3)
          m3 = jnp.maximum(m3, t3)
          m4 = jnp.maximum(m4, t4)
      g = jnp.maximum(g, _splat_f32(jnp.min(m4)))

    @plsc.parallel_loop(0, _NGRP)
    def _pass_a(grp):
      spn = grp // 8
      ri = grp % 8
      cb = spn * _GCOLS
      accs = [cur[ri, pl.ds(cb + 16 * a, 16)] for a in range(4)]
      for t in range(_SPAN):
        for kk in range(8):
          if t == 0 and kk < 4:
            continue
          a = kk % 4
          accs[a] = jnp.maximum(accs[a],
                                cur[ri, pl.ds(cb + t * 128 + 16 * kk, 16)])
      acc = jnp.maximum(jnp.maximum(accs[0], accs[1]),
                        jnp.maximum(accs[2], accs[3]))
      gm[grp] = acc

    def group(gidx, carry):
      qcnt, g = carry
      spn = gidx // 8
      ri = gidx % 8
      u_ = uref[pl.ds(16 * ri, 16)]
      anyp = jnp.any(gm[gidx] + u_ > g)

      def slow(qcnt, g):
        cb = spn * _GCOLS
        ivb = _splat_i32((rb * 8 + ri) * _VOCAB + col0 + w * _WCOLS + cb) + iota

        def tile_body(t, qcnt):
          for kk in range(8):
            off = 16 * kk
            x = cur[ri, pl.ds(cb + t * 128 + off, 16)] + u_
            m = x > g
            rows = jnp.minimum(qcnt, _QROWS - 1)
            plsc.store_scatter(qv, [rows, iota], x, mask=m)
            plsc.store_scatter(qi, [rows, iota], ivb + t * 128 + off, mask=m)
            qcnt = qcnt + jnp.where(m, one, zero)
          return qcnt

        qcnt = lax.fori_loop(0, _SPAN, tile_body, qcnt)
        return lax.cond(jnp.max(qcnt) > _QTRIG, compact_q, keep2, qcnt, g)

      return lax.cond(anyp, slow, keep2, qcnt, g)

    qcnt, g = lax.fori_loop(0, _NGRP, group, (qcnt, g))

  pltpu.async_copy(lp.at[pl.ds(rowbase8, 8), pl.ds(99968, 32)],
                   tailb, sems[0]).wait()
  for ri in range(8):
    u_ = uref[pl.ds(16 * ri, 16)]
    for kk in range(2):
      x = tailb[ri, pl.ds(16 * kk, 16)] + u_
      m = (x > g) & is_q0
      rows = jnp.minimum(qcnt, _QROWS - 1)
      plsc.store_scatter(qv, [rows, iota], x, mask=m)
      plsc.store_scatter(
          qi, [rows, iota],
          _splat_i32((rb * 8 + ri) * _VOCAB + 99968 + 16 * kk) + iota, mask=m)
      qcnt = qcnt + jnp.where(m, one, zero)

  maxq = jnp.max(qcnt)

  def flat_body(j, cnt):
    v = qv[j]
    ii = qi[j]
    m = j < qcnt
    plsc.store_compressed(candv.at[pl.ds(cnt, 16)], v, mask=m)
    plsc.store_compressed(candi.at[pl.ds(cnt, 16)], ii, mask=m)
    return cnt + jnp.max(plsc.all_reduce_population_count(m))

  cnt = lax.fori_loop(0, maxq, flat_body, jnp.int32(0))

  def compact(cnt, g2):
    candv[pl.ds(cnt, 16)] = _splat_f32(_NINF)
    nv = cnt // 16

    def lb_body(jj, acc):
      v = candv[pl.ds(jj * 16, 16)]
      sk, _ = plsc.sort_key_val(v, v)
      return jnp.minimum(acc, sk)

    acc = lax.fori_loop(0, nv, lb_body, _splat_f32(jnp.inf))
    lb = _splat_f32(lane12(acc))
    nv2 = (cnt + 15) // 16

    def f_body(jj, nc):
      v = candv[pl.ds(jj * 16, 16)]
      ii = candi[pl.ds(jj * 16, 16)]
      m = v >= lb
      plsc.store_compressed(candv.at[pl.ds(nc, 16)], v, mask=m)
      plsc.store_compressed(candi.at[pl.ds(nc, 16)], ii, mask=m)
      return nc + jnp.max(plsc.all_reduce_population_count(m))

    nc = lax.fori_loop(0, nv2, f_body, jnp.int32(0))
    return nc, g2

  def keepc(cnt, g2):
    return cnt, g2

  for _ in range(2):
    cnt, g = lax.cond(cnt > _OUTC, compact, keepc, cnt, g)

  for jj in range(_OUTC // 16):
    lanes = _splat_i32(16 * jj) + iota
    kp = lanes < cnt
    v = candv[pl.ds(16 * jj, 16)]
    ii = candi[pl.ds(16 * jj, 16)]
    candv[pl.ds(16 * jj, 16)] = jnp.where(kp, v, _splat_f32(_NINF))
    candi[pl.ds(16 * jj, 16)] = jnp.where(kp, ii, _splat_i32(_IMAX))

  m0 = iota == 0

  def ext_body(step, _):
    mv = _splat_f32(_NINF)
    for jj in range(_OUTC // 16):
      mv = jnp.maximum(mv, candv[pl.ds(16 * jj, 16)])
    ms = jnp.max(mv)
    mi = _splat_i32(_IMAX)
    for jj in range(_OUTC // 16):
      v = candv[pl.ds(16 * jj, 16)]
      ii = candi[pl.ds(16 * jj, 16)]
      mi = jnp.minimum(mi, jnp.where(v == ms, ii, _IMAX))
    ci = jnp.min(mi)
    for jj in range(_OUTC // 16):
      v = candv[pl.ds(16 * jj, 16)]
      ii = candi[pl.ds(16 * jj, 16)]
      candv[pl.ds(16 * jj, 16)] = jnp.where((v == ms) & (ii == ci),
                                            _splat_f32(_NINF), v)
    plsc.store_scatter(sortedv, [_splat_i32(step)], _splat_f32(ms), mask=m0)
    plsc.store_scatter(sortedi, [_splat_i32(step)], _splat_i32(ci), mask=m0)
    return 0

  lax.fori_loop(0, _BEAM, ext_body, 0)
  pltpu.sync_copy(sortedv, outv.at[pl.ds(wid * _BEAM, _BEAM)])
  pltpu.sync_copy(sortedi, outi.at[pl.ds(wid * _BEAM, _BEAM)])


def _merge_body(v_ref, i_ref, os_ref, ot_ref, oo_ref):
  vals0 = v_ref[...]
  idxs = i_ref[...]
  col = lax.broadcasted_iota(jnp.int32, (1, 128), 1)

  def body(i, carry):
    vals, sa, ta, oa = carry
    m = jnp.max(vals)
    sel = vals == m
    ci = jnp.min(jnp.where(sel, idxs, _IMAX))
    vals = jnp.where(sel & (idxs == ci), _NINF, vals)
    sa = jnp.where(col == i, m, sa)
    ta = jnp.where(col == i, ci % _VOCAB, ta)
    oa = jnp.where(col == i, ci // _VOCAB, oa)
    return vals, sa, ta, oa

  init = (vals0,
          jnp.full((1, 128), _NINF, jnp.float32),
          jnp.zeros((1, 128), jnp.int32),
          jnp.zeros((1, 128), jnp.int32))
  _, sa, ta, oa = lax.fori_loop(0, _BEAM, body, init)
  os_ref[...] = sa
  ot_ref[...] = ta
  oo_ref[...] = oa


def _sc_scan(lp, sp1, fp1):
  mesh = plsc.VectorSubcoreMesh(core_axis_name="c", subcore_axis_name="s",
                                num_cores=_NC, num_subcores=_NS)
  f = pl.kernel(
      _sc_scan_body,
      out_type=(jax.ShapeDtypeStruct((_NW * _BEAM,), jnp.float32),
                jax.ShapeDtypeStruct((_NW * _BEAM,), jnp.int32)),
      mesh=mesh,
      compiler_params=pltpu.CompilerParams(needs_layout_passes=False,
                                           use_tc_tiling_on_sc=True),
      scratch_types=[
          pltpu.VMEM((8, _WCOLS), jnp.float32),
          pltpu.VMEM((8, _WCOLS), jnp.float32),
          pltpu.VMEM((8, 32), jnp.float32),
          pltpu.VMEM((_NGRP, 16), jnp.float32),
          pltpu.VMEM((128,), jnp.float32),
          pltpu.VMEM((128,), jnp.float32),
          pltpu.VMEM((128,), jnp.int32),
          pltpu.VMEM((_QROWS, 16), jnp.float32),
          pltpu.VMEM((_QROWS, 16), jnp.int32),
          pltpu.VMEM((_FLAT,), jnp.float32),
          pltpu.VMEM((_FLAT,), jnp.int32),
          pltpu.VMEM((_BEAM,), jnp.float32),
          pltpu.VMEM((_BEAM,), jnp.int32),
          pltpu.SemaphoreType.DMA,
          pltpu.SemaphoreType.DMA,
      ],
  )
  return f(lp, sp1, fp1)


def _merge(cand_v, cand_i):
  return pl.pallas_call(
      _merge_body,
      out_shape=(jax.ShapeDtypeStruct((1, 128), jnp.float32),
                 jax.ShapeDtypeStruct((1, 128), jnp.int32),
                 jax.ShapeDtypeStruct((1, 128), jnp.int32)),
  )(cand_v, cand_i)


def kernel(lprobs, scores, finished):
  sp1 = jnp.broadcast_to(scores.reshape(_BEAM, 1).astype(jnp.float32),
                         (_BEAM, 16)).reshape(-1)
  fp1 = jnp.broadcast_to(finished.astype(jnp.int32).reshape(_BEAM, 1),
                         (_BEAM, 16)).reshape(-1)
  cand_v, cand_i = _sc_scan(lprobs, sp1, fp1)
  ts, tok, order = _merge(cand_v.reshape(_NW // 2, 2 * _BEAM),
                          cand_i.reshape(_NW // 2, 2 * _BEAM))
  return ts[0, :_BEAM], tok[0, :_BEAM], order[0, :_BEAM]
